# baseline (speedup 1.0000x reference)
import functools

import jax
import jax.numpy as jnp
from jax import lax
from jax.experimental import pallas as pl
from jax.experimental.pallas import tpu as pltpu
from jax.experimental.pallas import tpu_sc as plsc

N = 10000
E = 320000
D = 128
DH = 64
DOUT = 64
G = 16

NC = 2
NS = 16
EPT = E // NS
CH = 128
NCHK = 157
PR = 160
NP = 10240
RPT = NP // NS
ZCH = 128
NZ = RPT // ZCH
DCH = PR // NC

_mesh = plsc.VectorSubcoreMesh(core_axis_name="c", subcore_axis_name="s",
                               num_cores=NC, num_subcores=NS)


@functools.partial(
    pl.kernel,
    out_type=jax.ShapeDtypeStruct((NC, NP, 16), jnp.float32),
    mesh=_mesh,
    scratch_types=[
        pltpu.VMEM_SHARED((NP, 16), jnp.float32),
        pltpu.VMEM((DCH, CH), jnp.int32),
        pltpu.VMEM((RPT, 16), jnp.float32),
        pltpu.VMEM((CH, 16), jnp.float32),
        pltpu.SemaphoreType.DMA,
        pltpu.SemaphoreType.DMA,
        pltpu.SemaphoreType.DMA,
        pltpu.SemaphoreType.DMA,
    ],
)
def _sc_degree(idx_hbm, out_hbm, acc_sh, dst_v, stage_v, ones_v,
               dsem0, dsem1, dsem2, dsem3):
    c = lax.axis_index("c")
    s = lax.axis_index("s")
    base = s * RPT

    def _fill_row(r, val, ref):
        ref[r, :] = jnp.full((16,), val, jnp.float32)

    lax.fori_loop(0, RPT, lambda r, _: (_fill_row(r, 0.0, stage_v), 0)[1], 0)
    lax.fori_loop(0, CH, lambda r, _: (_fill_row(r, 1.0, ones_v), 0)[1], 0)
    pltpu.sync_copy(stage_v, acc_sh.at[pl.ds(base, RPT)])
    plsc.subcore_barrier()

    pltpu.sync_copy(idx_hbm.at[s, pl.ds(c * DCH, DCH)], dst_v)

    def _dscat(j, sem):
        return pltpu.make_async_copy(ones_v, acc_sh.at[dst_v.at[j]], sem)

    sems = (dsem0, dsem1, dsem2, dsem3)

    def _chunk(k, _):
        j = k * 4
        for t in range(4):
            @pl.when(j + t >= 4)
            def _():
                _dscat(j + t - 4, sems[t]).wait()

            _dscat(j + t, sems[t]).start(add=True)
        return 0

    lax.fori_loop(0, DCH // 4, _chunk, 0)
    for t in range(4):
        _dscat(DCH - 4 + t, sems[t]).wait()
    plsc.subcore_barrier()

    pltpu.sync_copy(acc_sh.at[pl.ds(base, RPT)], stage_v)
    pltpu.sync_copy(stage_v, out_hbm.at[c, pl.ds(base, RPT)])


@functools.partial(
    pl.kernel,
    out_type=jax.ShapeDtypeStruct((NC, NP, DH), jnp.float32),
    mesh=_mesh,
    scratch_types=[
        pltpu.VMEM_SHARED((NP, DH), jnp.float32),
        pltpu.VMEM((2 * NCHK, CH), jnp.int32),
        pltpu.VMEM((CH, DH), jnp.float32),
        pltpu.VMEM((CH, DH), jnp.float32),
        pltpu.VMEM((CH, DH), jnp.float32),
        pltpu.SemaphoreType.DMA,
        pltpu.SemaphoreType.DMA,
        pltpu.SemaphoreType.DMA,
        pltpu.SemaphoreType.DMA,
        pltpu.SemaphoreType.DMA,
        pltpu.SemaphoreType.DMA,
    ],
    compiler_params=pltpu.CompilerParams(use_tc_tiling_on_sc=False),
)
def _sc_scatter(xs_hbm, idx_hbm, out_hbm, acc_sh, idx_v,
                rows0, rows1, rows2, gsem0, gsem1, gsem2,
                ssem0, ssem1, ssem2):
    c = lax.axis_index("c")
    s = lax.axis_index("s")
    base = s * RPT

    def _zrow(r, _):
        for k in range(DH // 16):
            rows0[r, pl.ds(k * 16, 16)] = jnp.zeros((16,), jnp.float32)
        return 0

    lax.fori_loop(0, ZCH, _zrow, 0)
    for k in range(NZ):
        pltpu.sync_copy(rows0, acc_sh.at[pl.ds(base + k * ZCH, ZCH)])
    plsc.subcore_barrier()

    pltpu.sync_copy(idx_hbm.at[c, s], idx_v)

    def _gather(j, buf, sem):
        return pltpu.make_async_copy(xs_hbm.at[idx_v.at[j]], buf, sem)

    def _scatter(j, buf, sem):
        return pltpu.make_async_copy(buf, acc_sh.at[idx_v.at[NCHK + j]], sem)

    _DIAG_NO_SCATTER = True

    def _iter(j, cur_rows, cur_g, cur_s, oth_rows, oth_g, oth_s,
              first_guard):
        _gather(j, cur_rows, cur_g).wait()
        if not _DIAG_NO_SCATTER:
            _scatter(j, cur_rows, cur_s).start(add=True)

            if first_guard:
                @pl.when(j >= 1)
                def _():
                    _scatter(j - 1, oth_rows, oth_s).wait()
            else:
                _scatter(j - 1, oth_rows, oth_s).wait()

        @pl.when(j + 2 < NCHK)
        def _():
            _gather(j + 2, oth_rows, oth_g).start()

    _gather(0, rows0, gsem0).start()
    _gather(1, rows1, gsem1).start()

    def _step(k, _):
        j0 = k * 3
        _iter(j0, rows0, gsem0, ssem0, rows2, gsem2, ssem2, True)

        @pl.when(j0 + 1 < NCHK)
        def _():
            _iter(j0 + 1, rows1, gsem1, ssem1, rows0, gsem0, ssem0, False)

        @pl.when(j0 + 2 < NCHK)
        def _():
            _iter(j0 + 2, rows2, gsem2, ssem2, rows1, gsem1, ssem1, False)

        return 0

    lax.fori_loop(0, (NCHK + 2) // 3, _step, 0)
    if not _DIAG_NO_SCATTER:
        _scatter(NCHK - 1, rows0, ssem0).wait()
    plsc.subcore_barrier()

    for k in range(NZ):
        pltpu.sync_copy(acc_sh.at[pl.ds(base + k * ZCH, ZCH)], rows0)
        pltpu.sync_copy(rows0, out_hbm.at[c, pl.ds(base + k * ZCH, ZCH)])


_BR = 1000


def _dinv_from(deg_ref):
    return lax.rsqrt(deg_ref[0, :, 0:1] + deg_ref[1, :, 0:1] + 1.0)


def _tc_scale_matmul_body(deg_ref, x_ref, w_ref, out_ref):
    dinv = _dinv_from(deg_ref)
    out_ref[...] = jnp.dot(x_ref[...], w_ref[...],
                           preferred_element_type=jnp.float32) * dinv


def _tc_scale_matmul(degp, x, w):
    return pl.pallas_call(
        _tc_scale_matmul_body,
        grid=(N // _BR,),
        in_specs=[
            pl.BlockSpec((NC, _BR, 16), lambda i: (0, i, 0)),
            pl.BlockSpec((_BR, D), lambda i: (i, 0)),
            pl.BlockSpec((D, D), lambda i: (0, 0)),
        ],
        out_specs=pl.BlockSpec((_BR, D), lambda i: (i, 0)),
        out_shape=jax.ShapeDtypeStruct((N, D), jnp.float32),
    )(degp, x, w)


def _tc_layer2_body(deg_ref, sp_ref, xs_ref, b_ref, w_ref, out_ref):
    dinv = _dinv_from(deg_ref)
    seg = jnp.concatenate([sp_ref[0], sp_ref[1]], axis=-1)
    h = dinv * (seg + xs_ref[...]) + b_ref[...]
    h = jnp.maximum(h, 0.0)
    out_ref[...] = jnp.dot(h, w_ref[...],
                           preferred_element_type=jnp.float32) * dinv


def _tc_layer2(degp, sparts, xs, b, w):
    return pl.pallas_call(
        _tc_layer2_body,
        grid=(N // _BR,),
        in_specs=[
            pl.BlockSpec((NC, _BR, 16), lambda i: (0, i, 0)),
            pl.BlockSpec((NC, _BR, DH), lambda i: (0, i, 0)),
            pl.BlockSpec((_BR, D), lambda i: (i, 0)),
            pl.BlockSpec((1, D), lambda i: (0, 0)),
            pl.BlockSpec((D, D), lambda i: (0, 0)),
        ],
        out_specs=pl.BlockSpec((_BR, D), lambda i: (i, 0)),
        out_shape=jax.ShapeDtypeStruct((N, D), jnp.float32),
    )(degp, sparts, xs, b, w)


def _tc_final_body(deg_ref, sp_ref, xs_ref, b_ref, batch_ref, wfc_ref,
                   bfc_ref, out_ref):
    dinv = lax.rsqrt(deg_ref[0, :N, 0:1] + deg_ref[1, :N, 0:1] + 1.0)
    seg = jnp.concatenate([sp_ref[0, :N], sp_ref[1, :N]], axis=-1)
    h2 = dinv * (seg + xs_ref[...]) + b_ref[...]
    gids = lax.broadcasted_iota(jnp.int32, (G, N), 0)
    oh = (gids == batch_ref[...]).astype(jnp.float32)
    psum = jnp.dot(oh, h2, preferred_element_type=jnp.float32)
    cnt = jnp.sum(oh, axis=1, keepdims=True)
    pooled = psum / jnp.maximum(cnt, 1.0)
    out_ref[...] = jnp.dot(pooled, wfc_ref[...],
                           preferred_element_type=jnp.float32) + bfc_ref[...]


def _tc_final(degp, sparts, xs, b, batchrow, wfc, bfc):
    return pl.pallas_call(
        _tc_final_body,
        out_shape=jax.ShapeDtypeStruct((G, DOUT), jnp.float32),
    )(degp, sparts, xs, b, batchrow, wfc, bfc)


def _pack_indices(src, dst):
    pad = NCHK * CH - EPT
    s2 = jnp.pad((src * 2).reshape(NS, EPT),
                 ((0, 0), (0, pad))).reshape(NS, NCHK, CH)
    d2 = jnp.pad(dst.reshape(NS, EPT), ((0, 0), (0, pad)),
                 constant_values=NP - 1).reshape(NS, NCHK, CH)
    idx = jnp.stack([jnp.concatenate([s2, d2], axis=1),
                     jnp.concatenate([s2 + 1, d2], axis=1)])
    deg_idx = jnp.pad(d2, ((0, 0), (0, PR - NCHK), (0, 0)),
                      constant_values=NP - 1)
    return idx, deg_idx


def kernel(x, edge_index, batch, W1, b1, W2, b2, Wfc, bfc):
    idx, deg_idx = _pack_indices(edge_index[0], edge_index[1])
    batchrow = batch.reshape(1, N)

    degp = _sc_degree(deg_idx)

    xs1 = _tc_scale_matmul(degp, x, W1)
    s1 = _sc_scatter(xs1.reshape(2 * N, DH), idx)
    xs2 = _tc_layer2(degp, s1, xs1, b1.reshape(1, D), W2)
    s2 = _sc_scatter(xs2.reshape(2 * N, DH), idx)
    return _tc_final(degp, s2, xs2, b2.reshape(1, D), batchrow, Wfc, bfc)


# speedup vs baseline: 1.2706x; 1.2706x over previous
import functools

import jax
import jax.numpy as jnp
from jax import lax
from jax.experimental import pallas as pl
from jax.experimental.pallas import tpu as pltpu
from jax.experimental.pallas import tpu_sc as plsc

N = 10000
E = 320000
D = 128
DH = 64
DOUT = 64
G = 16

NC = 2
NS = 16
EPT = E // NS
CH = 128
NCHK = 157
PR = 160
NP = 10240
RPT = NP // NS
ZCH = 128
NZ = RPT // ZCH
DCH = PR // NC

_mesh = plsc.VectorSubcoreMesh(core_axis_name="c", subcore_axis_name="s",
                               num_cores=NC, num_subcores=NS)


@functools.partial(
    pl.kernel,
    out_type=jax.ShapeDtypeStruct((NC, NP, 16), jnp.float32),
    mesh=_mesh,
    scratch_types=[
        pltpu.VMEM_SHARED((NP, 16), jnp.float32),
        pltpu.VMEM((DCH, CH), jnp.int32),
        pltpu.VMEM((RPT, 16), jnp.float32),
        pltpu.VMEM((CH, 16), jnp.float32),
        pltpu.SemaphoreType.DMA,
        pltpu.SemaphoreType.DMA,
        pltpu.SemaphoreType.DMA,
        pltpu.SemaphoreType.DMA,
    ],
)
def _sc_degree(idx_hbm, out_hbm, acc_sh, dst_v, stage_v, ones_v,
               dsem0, dsem1, dsem2, dsem3):
    c = lax.axis_index("c")
    s = lax.axis_index("s")
    base = s * RPT

    def _fill_row(r, val, ref):
        ref[r, :] = jnp.full((16,), val, jnp.float32)

    lax.fori_loop(0, RPT, lambda r, _: (_fill_row(r, 0.0, stage_v), 0)[1], 0)
    lax.fori_loop(0, CH, lambda r, _: (_fill_row(r, 1.0, ones_v), 0)[1], 0)
    pltpu.sync_copy(stage_v, acc_sh.at[pl.ds(base, RPT)])
    plsc.subcore_barrier()

    pltpu.sync_copy(idx_hbm.at[s, pl.ds(c * DCH, DCH)], dst_v)

    def _dscat(j, sem):
        return pltpu.make_async_copy(ones_v, acc_sh.at[dst_v.at[j]], sem)

    sems = (dsem0, dsem1, dsem2, dsem3)

    def _chunk(k, _):
        j = k * 4
        for t in range(4):
            @pl.when(j + t >= 4)
            def _():
                _dscat(j + t - 4, sems[t]).wait()

            _dscat(j + t, sems[t]).start(add=True)
        return 0

    lax.fori_loop(0, DCH // 4, _chunk, 0)
    for t in range(4):
        _dscat(DCH - 4 + t, sems[t]).wait()
    plsc.subcore_barrier()

    pltpu.sync_copy(acc_sh.at[pl.ds(base, RPT)], stage_v)
    pltpu.sync_copy(stage_v, out_hbm.at[c, pl.ds(base, RPT)])


@functools.partial(
    pl.kernel,
    out_type=jax.ShapeDtypeStruct((NC, NP, DH), jnp.bfloat16),
    mesh=_mesh,
    scratch_types=[
        pltpu.VMEM_SHARED((NP, DH), jnp.bfloat16),
        pltpu.VMEM((2 * NCHK, CH), jnp.int32),
        pltpu.VMEM((CH, DH), jnp.bfloat16),
        pltpu.VMEM((CH, DH), jnp.bfloat16),
        pltpu.VMEM((CH, DH), jnp.bfloat16),
        pltpu.SemaphoreType.DMA,
        pltpu.SemaphoreType.DMA,
        pltpu.SemaphoreType.DMA,
        pltpu.SemaphoreType.DMA,
        pltpu.SemaphoreType.DMA,
        pltpu.SemaphoreType.DMA,
    ],
    compiler_params=pltpu.CompilerParams(use_tc_tiling_on_sc=False),
)
def _sc_scatter(xs_hbm, idx_hbm, out_hbm, acc_sh, idx_v,
                rows0, rows1, rows2, gsem0, gsem1, gsem2,
                ssem0, ssem1, ssem2):
    c = lax.axis_index("c")
    s = lax.axis_index("s")
    base = s * RPT

    def _zrow(r, _):
        for k in range(DH // 32):
            rows0[r, pl.ds(k * 32, 32)] = jnp.zeros((32,), jnp.bfloat16)
        return 0

    lax.fori_loop(0, ZCH, _zrow, 0)
    for k in range(NZ):
        pltpu.sync_copy(rows0, acc_sh.at[pl.ds(base + k * ZCH, ZCH)])
    plsc.subcore_barrier()

    pltpu.sync_copy(idx_hbm.at[c, s], idx_v)

    def _gather(j, buf, sem):
        return pltpu.make_async_copy(xs_hbm.at[idx_v.at[j]], buf, sem)

    def _scatter(j, buf, sem):
        return pltpu.make_async_copy(buf, acc_sh.at[idx_v.at[NCHK + j]], sem)

    def _iter(j, cur_rows, cur_g, cur_s, oth_rows, oth_g, oth_s,
              first_guard):
        _gather(j, cur_rows, cur_g).wait()
        _scatter(j, cur_rows, cur_s).start(add=True)

        if first_guard:
            @pl.when(j >= 1)
            def _():
                _scatter(j - 1, oth_rows, oth_s).wait()
        else:
            _scatter(j - 1, oth_rows, oth_s).wait()

        @pl.when(j + 2 < NCHK)
        def _():
            _gather(j + 2, oth_rows, oth_g).start()

    _gather(0, rows0, gsem0).start()
    _gather(1, rows1, gsem1).start()

    def _step(k, _):
        j0 = k * 3
        _iter(j0, rows0, gsem0, ssem0, rows2, gsem2, ssem2, True)

        @pl.when(j0 + 1 < NCHK)
        def _():
            _iter(j0 + 1, rows1, gsem1, ssem1, rows0, gsem0, ssem0, False)

        @pl.when(j0 + 2 < NCHK)
        def _():
            _iter(j0 + 2, rows2, gsem2, ssem2, rows1, gsem1, ssem1, False)

        return 0

    lax.fori_loop(0, (NCHK + 2) // 3, _step, 0)
    _scatter(NCHK - 1, rows0, ssem0).wait()
    plsc.subcore_barrier()

    for k in range(NZ):
        pltpu.sync_copy(acc_sh.at[pl.ds(base + k * ZCH, ZCH)], rows0)
        pltpu.sync_copy(rows0, out_hbm.at[c, pl.ds(base + k * ZCH, ZCH)])


_BR = 1000


def _dinv_from(deg_ref):
    return lax.rsqrt(deg_ref[0, :, 0:1] + deg_ref[1, :, 0:1] + 1.0)


def _tc_scale_matmul_body(deg_ref, x_ref, w_ref, out_ref):
    dinv = _dinv_from(deg_ref)
    out_ref[...] = (jnp.dot(x_ref[...], w_ref[...],
                            preferred_element_type=jnp.float32)
                    * dinv).astype(jnp.bfloat16)


def _tc_scale_matmul(degp, x, w):
    return pl.pallas_call(
        _tc_scale_matmul_body,
        grid=(N // _BR,),
        in_specs=[
            pl.BlockSpec((NC, _BR, 16), lambda i: (0, i, 0)),
            pl.BlockSpec((_BR, D), lambda i: (i, 0)),
            pl.BlockSpec((D, D), lambda i: (0, 0)),
        ],
        out_specs=pl.BlockSpec((_BR, D), lambda i: (i, 0)),
        out_shape=jax.ShapeDtypeStruct((N, D), jnp.bfloat16),
    )(degp, x, w)


def _tc_layer2_body(deg_ref, sp_ref, xs_ref, b_ref, w_ref, out_ref):
    dinv = _dinv_from(deg_ref)
    seg = jnp.concatenate([sp_ref[0], sp_ref[1]],
                          axis=-1).astype(jnp.float32)
    h = dinv * (seg + xs_ref[...].astype(jnp.float32)) + b_ref[...]
    h = jnp.maximum(h, 0.0)
    out_ref[...] = (jnp.dot(h, w_ref[...],
                            preferred_element_type=jnp.float32)
                    * dinv).astype(jnp.bfloat16)


def _tc_layer2(degp, sparts, xs, b, w):
    return pl.pallas_call(
        _tc_layer2_body,
        grid=(N // _BR,),
        in_specs=[
            pl.BlockSpec((NC, _BR, 16), lambda i: (0, i, 0)),
            pl.BlockSpec((NC, _BR, DH), lambda i: (0, i, 0)),
            pl.BlockSpec((_BR, D), lambda i: (i, 0)),
            pl.BlockSpec((1, D), lambda i: (0, 0)),
            pl.BlockSpec((D, D), lambda i: (0, 0)),
        ],
        out_specs=pl.BlockSpec((_BR, D), lambda i: (i, 0)),
        out_shape=jax.ShapeDtypeStruct((N, D), jnp.bfloat16),
    )(degp, sparts, xs, b, w)


def _tc_final_body(deg_ref, sp_ref, xs_ref, b_ref, batch_ref, wfc_ref,
                   bfc_ref, out_ref):
    dinv = lax.rsqrt(deg_ref[0, :N, 0:1] + deg_ref[1, :N, 0:1] + 1.0)
    seg = jnp.concatenate([sp_ref[0, :N], sp_ref[1, :N]],
                          axis=-1).astype(jnp.float32)
    h2 = dinv * (seg + xs_ref[...].astype(jnp.float32)) + b_ref[...]
    gids = lax.broadcasted_iota(jnp.int32, (G, N), 0)
    oh = (gids == batch_ref[...]).astype(jnp.float32)
    psum = jnp.dot(oh, h2, preferred_element_type=jnp.float32)
    cnt = jnp.sum(oh, axis=1, keepdims=True)
    pooled = psum / jnp.maximum(cnt, 1.0)
    out_ref[...] = jnp.dot(pooled, wfc_ref[...],
                           preferred_element_type=jnp.float32) + bfc_ref[...]


def _tc_final(degp, sparts, xs, b, batchrow, wfc, bfc):
    return pl.pallas_call(
        _tc_final_body,
        out_shape=jax.ShapeDtypeStruct((G, DOUT), jnp.float32),
    )(degp, sparts, xs, b, batchrow, wfc, bfc)


def _pack_indices(src, dst):
    pad = NCHK * CH - EPT
    s2 = jnp.pad((src * 2).reshape(NS, EPT),
                 ((0, 0), (0, pad))).reshape(NS, NCHK, CH)
    d2 = jnp.pad(dst.reshape(NS, EPT), ((0, 0), (0, pad)),
                 constant_values=NP - 1).reshape(NS, NCHK, CH)
    idx = jnp.stack([jnp.concatenate([s2, d2], axis=1),
                     jnp.concatenate([s2 + 1, d2], axis=1)])
    deg_idx = jnp.pad(d2, ((0, 0), (0, PR - NCHK), (0, 0)),
                      constant_values=NP - 1)
    return idx, deg_idx


def kernel(x, edge_index, batch, W1, b1, W2, b2, Wfc, bfc):
    idx, deg_idx = _pack_indices(edge_index[0], edge_index[1])
    batchrow = batch.reshape(1, N)

    degp = _sc_degree(deg_idx)

    xs1 = _tc_scale_matmul(degp, x, W1)
    s1 = _sc_scatter(xs1.reshape(2 * N, DH), idx)
    xs2 = _tc_layer2(degp, s1, xs1, b1.reshape(1, D), W2)
    s2 = _sc_scatter(xs2.reshape(2 * N, DH), idx)
    return _tc_final(degp, s2, xs2, b2.reshape(1, D), batchrow, Wfc, bfc)
